# Initial kernel scaffold; baseline (speedup 1.0000x reference)
import jax
import jax.numpy as jnp
from jax.experimental import pallas as pl


def kernel(logits, labels, batch_indices, label_ids):
    raise NotImplementedError("write your pallas kernel here")


# speedup vs baseline: 5.9793x; 5.9793x over previous
import functools

import jax
import jax.numpy as jnp
from jax import lax
from jax.experimental import pallas as pl
from jax.experimental.pallas import tpu as pltpu
from jax.experimental.pallas import tpu_sc as plsc

_B = 4096
_L = 50
_N = _B * _L
_NW = 32
_CHUNK = _N // _NW
_NV = _CHUNK // 16
_NPACK = _N // 4
_TBL = 2 * _B

_MARGIN = 0.5
_SCALE_LOSS = 10.0
_TEMP_BASE = 10.0
_SEP_W = 0.5


def _sc_body(logits_hbm, bidx_hbm, lidx_hbm, packed_hbm,
             tbl_out, part_out,
             x_v, bi_v, li_v, pk_v, tbl_v, part_v):
    wid = lax.axis_index("s") * 2 + lax.axis_index("c")
    base = pl.multiple_of(wid * _CHUNK, _CHUNK)

    pltpu.sync_copy(logits_hbm.at[pl.ds(base, _CHUNK)], x_v)
    pltpu.sync_copy(bidx_hbm.at[pl.ds(base, _CHUNK)], bi_v)
    pltpu.sync_copy(lidx_hbm.at[pl.ds(base, _CHUNK)], li_v)
    pltpu.sync_copy(packed_hbm, pk_v)

    neg_inf16 = jnp.full((16,), -jnp.inf, dtype=jnp.float32)

    def init_body(i, c):
        tbl_v[pl.ds(pl.multiple_of(i * 16, 16), 16)] = neg_inf16
        return c

    lax.fori_loop(0, _TBL // 16, init_body, 0)

    zero16 = jnp.zeros((16,), dtype=jnp.float32)

    def body(j, carry):
        xt, cnt, pr, nr = carry
        off = pl.multiple_of(j * 16, 16)
        x = x_v[pl.ds(off, 16)]
        bi = bi_v[pl.ds(off, 16)]
        li = li_v[pl.ds(off, 16)]
        flat = bi * _L + li - 1
        word = lax.shift_right_logical(flat, 2)
        g = plsc.load_gather(pk_v, [word])
        tb = lax.shift_right_logical(g, (flat & 3) * 8) & 1
        t = tb.astype(jnp.float32)
        key = bi + lax.shift_left(tb, 12)
        v = x * (1.0 - 2.0 * t)

        def wcond(m_i32):
            return jnp.max(m_i32) > 0

        def wbody(m_i32):
            m = m_i32 > 0
            cur = plsc.load_gather(tbl_v, [key], mask=m)
            plsc.store_scatter(tbl_v, [key], jnp.maximum(cur, v), mask=m)
            cur2 = plsc.load_gather(tbl_v, [key], mask=m)
            return jnp.where(m & (cur2 < v), 1, 0).astype(jnp.int32)

        lax.while_loop(wcond, wbody, jnp.ones((16,), dtype=jnp.int32))

        xt = xt + x * t
        cnt = cnt + t
        pr = pr + t * jnp.maximum(1.0 - x, 0.0)
        nr = nr + (1.0 - t) * jnp.maximum(x + 1.0, 0.0)
        return xt, cnt, pr, nr

    xt, cnt, pr, nr = lax.fori_loop(0, _NV, body, (zero16, zero16, zero16, zero16))

    part_v[pl.ds(0, 16)] = xt
    part_v[pl.ds(16, 16)] = cnt
    part_v[pl.ds(32, 16)] = pr
    part_v[pl.ds(48, 16)] = nr

    pltpu.sync_copy(tbl_v, tbl_out.at[wid])
    pltpu.sync_copy(part_v, part_out.at[wid])


_sc_call = functools.partial(
    pl.kernel,
    out_type=(
        jax.ShapeDtypeStruct((_NW, _TBL), jnp.float32),
        jax.ShapeDtypeStruct((_NW, 64), jnp.float32),
    ),
    mesh=plsc.VectorSubcoreMesh(core_axis_name="c", subcore_axis_name="s"),
    compiler_params=pltpu.CompilerParams(needs_layout_passes=False),
    scratch_types=[
        pltpu.VMEM((_CHUNK,), jnp.float32),
        pltpu.VMEM((_CHUNK,), jnp.int32),
        pltpu.VMEM((_CHUNK,), jnp.int32),
        pltpu.VMEM((_NPACK,), jnp.int32),
        pltpu.VMEM((_TBL,), jnp.float32),
        pltpu.VMEM((64,), jnp.float32),
    ],
)(_sc_body)


def _tc_body(x_ref, tbl_ref, part_ref, out_ref):
    x = x_ref[...]
    bce_base = jnp.sum(jnp.maximum(x, 0.0) + jnp.log1p(jnp.exp(-jnp.abs(x))))

    tbl = jnp.max(tbl_ref[...], axis=0, keepdims=True)
    max_neg = tbl[:, :_B]
    neg_min_pos = tbl[:, _B:]
    ninf = jnp.float32(-jnp.inf)
    valid = (max_neg > ninf) & (neg_min_pos > ninf)
    samp = jnp.where(valid, jnp.maximum(_MARGIN + max_neg + neg_min_pos, 0.0), 0.0)
    num_valid = jnp.sum(valid.astype(jnp.float32))
    samp_sum = jnp.sum(samp)

    p = part_ref[...]
    s_xt = jnp.sum(p[:, 0:16])
    pos_cnt = jnp.sum(p[:, 16:32])
    pr = jnp.sum(p[:, 32:48])
    nr = jnp.sum(p[:, 48:64])

    n_valid = jnp.float32(_N)
    neg_cnt = n_valid - pos_cnt
    bce_loss = (bce_base - s_xt) / n_valid * _SCALE_LOSS
    avg_valid = n_valid / jnp.maximum(num_valid, 1.0)
    temperature = _TEMP_BASE / jnp.maximum(avg_valid, 1.0)
    contrastive = samp_sum * temperature
    sep = jnp.where(pos_cnt > 0.0, pr / jnp.maximum(pos_cnt, 1.0), 0.0)
    sep = sep + jnp.where(neg_cnt > 0.0, nr / jnp.maximum(neg_cnt, 1.0), 0.0)
    out_ref[0, 0] = bce_loss + contrastive + _SEP_W * sep


def kernel(logits, labels, batch_indices, label_ids):
    packed = lax.bitcast_convert_type(
        labels.reshape(_NPACK, 4).astype(jnp.int8), jnp.int32)
    x_flat = logits.reshape(-1)

    tbl_all, part_all = _sc_call(x_flat, batch_indices, label_ids, packed)

    x2d = x_flat.reshape(_N // 128, 128)
    out = pl.pallas_call(
        _tc_body,
        out_shape=jax.ShapeDtypeStruct((1, 1), jnp.float32),
        out_specs=pl.BlockSpec(memory_space=pltpu.SMEM),
    )(x2d, tbl_all, part_all)
    return out.reshape(())


# speedup vs baseline: 7.9267x; 1.3257x over previous
import functools

import jax
import jax.numpy as jnp
from jax import lax
from jax.experimental import pallas as pl
from jax.experimental.pallas import tpu as pltpu
from jax.experimental.pallas import tpu_sc as plsc

_B = 4096
_L = 50
_N = _B * _L
_NW = 32
_CHUNK = _N // _NW
_NV = _CHUNK // 16
_NPACK = _N // 4
_TBL = 2 * _B

_MARGIN = 0.5
_SCALE_LOSS = 10.0
_TEMP_BASE = 10.0
_SEP_W = 0.5


def _pack_body(lab_ref, out_ref):
    b0 = lab_ref[0:1024, :]
    b1 = lab_ref[1024:2048, :]
    b2 = lab_ref[2048:3072, :]
    b3 = lab_ref[3072:4096, :]
    out_ref[...] = b0 | (b1 << 8) | (b2 << 16) | (b3 << 24)


def _sc_body(logits_hbm, bidx_hbm, lidx_hbm, packed_hbm,
             tbl_out, part_out,
             x_v, bi_v, li_v, pk_v, tbl_v, part_v):
    wid = lax.axis_index("s") * 2 + lax.axis_index("c")
    base = pl.multiple_of(wid * _CHUNK, _CHUNK)

    pltpu.sync_copy(logits_hbm.at[pl.ds(base, _CHUNK)], x_v)
    pltpu.sync_copy(bidx_hbm.at[pl.ds(base, _CHUNK)], bi_v)
    pltpu.sync_copy(lidx_hbm.at[pl.ds(base, _CHUNK)], li_v)
    pltpu.sync_copy(packed_hbm, pk_v)

    neg_inf16 = jnp.full((16,), -jnp.inf, dtype=jnp.float32)

    def init_body(i, c):
        tbl_v[pl.ds(pl.multiple_of(i * 16, 16), 16)] = neg_inf16
        return c

    lax.fori_loop(0, _TBL // 16, init_body, 0)

    zero16 = jnp.zeros((16,), dtype=jnp.float32)

    def body(j, carry):
        xt, cnt, pr, nr = carry
        off = pl.multiple_of(j * 16, 16)
        x = x_v[pl.ds(off, 16)]
        bi = bi_v[pl.ds(off, 16)]
        li = li_v[pl.ds(off, 16)]
        word = (bi & 1023) * _L + li - 1
        g = plsc.load_gather(pk_v, [word])
        sh = lax.shift_left(lax.shift_right_logical(bi, 10), 3)
        tb = lax.shift_right_logical(g, sh) & 1
        t = tb.astype(jnp.float32)
        key = bi + lax.shift_left(tb, 12)
        v = x * (1.0 - 2.0 * t)

        def wcond(m_i32):
            return jnp.max(m_i32) > 0

        def wbody(m_i32):
            m = m_i32 > 0
            cur = plsc.load_gather(tbl_v, [key], mask=m)
            plsc.store_scatter(tbl_v, [key], jnp.maximum(cur, v), mask=m)
            cur2 = plsc.load_gather(tbl_v, [key], mask=m)
            return jnp.where(m & (cur2 < v), 1, 0).astype(jnp.int32)

        lax.while_loop(wcond, wbody, jnp.ones((16,), dtype=jnp.int32))

        xt = xt + x * t
        cnt = cnt + t
        pr = pr + t * jnp.maximum(1.0 - x, 0.0)
        nr = nr + (1.0 - t) * jnp.maximum(x + 1.0, 0.0)
        return xt, cnt, pr, nr

    xt, cnt, pr, nr = lax.fori_loop(0, _NV, body, (zero16, zero16, zero16, zero16))

    part_v[pl.ds(0, 16)] = xt
    part_v[pl.ds(16, 16)] = cnt
    part_v[pl.ds(32, 16)] = pr
    part_v[pl.ds(48, 16)] = nr

    pltpu.sync_copy(tbl_v, tbl_out.at[wid])
    pltpu.sync_copy(part_v, part_out.at[wid])


_sc_call = functools.partial(
    pl.kernel,
    out_type=(
        jax.ShapeDtypeStruct((_NW, _TBL), jnp.float32),
        jax.ShapeDtypeStruct((_NW, 64), jnp.float32),
    ),
    mesh=plsc.VectorSubcoreMesh(core_axis_name="c", subcore_axis_name="s"),
    compiler_params=pltpu.CompilerParams(needs_layout_passes=False),
    scratch_types=[
        pltpu.VMEM((_CHUNK,), jnp.float32),
        pltpu.VMEM((_CHUNK,), jnp.int32),
        pltpu.VMEM((_CHUNK,), jnp.int32),
        pltpu.VMEM((_NPACK,), jnp.int32),
        pltpu.VMEM((_TBL,), jnp.float32),
        pltpu.VMEM((64,), jnp.float32),
    ],
)(_sc_body)


def _tc_body(x_ref, tbl_ref, part_ref, out_ref):
    x = x_ref[...]
    bce_base = jnp.sum(jnp.maximum(x, 0.0) + jnp.log1p(jnp.exp(-jnp.abs(x))))

    tbl = jnp.max(tbl_ref[...], axis=0, keepdims=True)
    max_neg = tbl[:, :_B]
    neg_min_pos = tbl[:, _B:]
    ninf = jnp.float32(-jnp.inf)
    valid = (max_neg > ninf) & (neg_min_pos > ninf)
    samp = jnp.where(valid, jnp.maximum(_MARGIN + max_neg + neg_min_pos, 0.0), 0.0)
    num_valid = jnp.sum(valid.astype(jnp.float32))
    samp_sum = jnp.sum(samp)

    p = part_ref[...]
    s_xt = jnp.sum(p[:, 0:16])
    pos_cnt = jnp.sum(p[:, 16:32])
    pr = jnp.sum(p[:, 32:48])
    nr = jnp.sum(p[:, 48:64])

    n_valid = jnp.float32(_N)
    neg_cnt = n_valid - pos_cnt
    bce_loss = (bce_base - s_xt) / n_valid * _SCALE_LOSS
    avg_valid = n_valid / jnp.maximum(num_valid, 1.0)
    temperature = _TEMP_BASE / jnp.maximum(avg_valid, 1.0)
    contrastive = samp_sum * temperature
    sep = jnp.where(pos_cnt > 0.0, pr / jnp.maximum(pos_cnt, 1.0), 0.0)
    sep = sep + jnp.where(neg_cnt > 0.0, nr / jnp.maximum(neg_cnt, 1.0), 0.0)
    out_ref[0, 0] = bce_loss + contrastive + _SEP_W * sep


def kernel(logits, labels, batch_indices, label_ids):
    packed = pl.pallas_call(
        _pack_body,
        out_shape=jax.ShapeDtypeStruct((_B // 4, _L), jnp.int32),
    )(labels).reshape(-1)
    x_flat = logits.reshape(-1)

    tbl_all, part_all = _sc_call(x_flat, batch_indices, label_ids, packed)

    out = pl.pallas_call(
        _tc_body,
        out_shape=jax.ShapeDtypeStruct((1, 1), jnp.float32),
        out_specs=pl.BlockSpec(memory_space=pltpu.SMEM),
    )(x_flat, tbl_all, part_all)
    return out.reshape(())


# speedup vs baseline: 8.7886x; 1.1087x over previous
import functools

import jax
import jax.numpy as jnp
from jax import lax
from jax.experimental import pallas as pl
from jax.experimental.pallas import tpu as pltpu
from jax.experimental.pallas import tpu_sc as plsc

_B = 4096
_L = 50
_N = _B * _L
_NW = 32
_CHUNK = _N // _NW
_NV = _CHUNK // 16
_NPACK = _N // 4
_TBL = 2 * _B

_MARGIN = 0.5
_SCALE_LOSS = 10.0
_TEMP_BASE = 10.0
_SEP_W = 0.5


def _pack_body(lab_ref, out_ref):
    b0 = lab_ref[0:1024, :]
    b1 = lab_ref[1024:2048, :]
    b2 = lab_ref[2048:3072, :]
    b3 = lab_ref[3072:4096, :]
    out_ref[...] = b0 | (b1 << 8) | (b2 << 16) | (b3 << 24)


def _sc_body(logits_hbm, bidx_hbm, lidx_hbm, packed_hbm,
             tbl_out, part_out,
             x_v, bi_v, li_v, pk_v,
             tbl0, tbl1, tbl2, tbl3,
             key_s, v_s, m_s, part_v,
             sem):
    wid = lax.axis_index("s") * 2 + lax.axis_index("c")
    base = pl.multiple_of(wid * _CHUNK, _CHUNK)

    cx = pltpu.async_copy(logits_hbm.at[pl.ds(base, _CHUNK)], x_v, sem)
    cb = pltpu.async_copy(bidx_hbm.at[pl.ds(base, _CHUNK)], bi_v, sem)
    cl = pltpu.async_copy(lidx_hbm.at[pl.ds(base, _CHUNK)], li_v, sem)
    cp = pltpu.async_copy(packed_hbm, pk_v, sem)

    neg_inf16 = jnp.full((16,), -jnp.inf, dtype=jnp.float32)

    def init_body(i, c):
        off = pl.multiple_of(i * 16, 16)
        tbl0[pl.ds(off, 16)] = neg_inf16
        tbl1[pl.ds(off, 16)] = neg_inf16
        tbl2[pl.ds(off, 16)] = neg_inf16
        tbl3[pl.ds(off, 16)] = neg_inf16
        return c

    lax.fori_loop(0, _TBL // 16, init_body, 0)

    cx.wait()
    cb.wait()
    cl.wait()
    cp.wait()

    zero16 = jnp.zeros((16,), dtype=jnp.float32)
    izero16 = jnp.zeros((16,), dtype=jnp.int32)
    tables = (tbl0, tbl1, tbl2, tbl3)

    def body(i, carry):
        xt, cnt, pr, nr, pend = carry
        for q in range(4):
            off = pl.multiple_of(i * 64, 64) + q * 16
            x = x_v[pl.ds(off, 16)]
            bi = bi_v[pl.ds(off, 16)]
            li = li_v[pl.ds(off, 16)]
            word = (bi & 1023) * _L + li - 1
            g = plsc.load_gather(pk_v, [word])
            sh = lax.shift_left(lax.shift_right_logical(bi, 10), 3)
            tb = lax.shift_right_logical(g, sh) & 1
            t = tb.astype(jnp.float32)
            key = bi + lax.shift_left(tb, 12)
            v = x * (1.0 - 2.0 * t)

            tbl = tables[q]
            cur = plsc.load_gather(tbl, [key])
            plsc.store_scatter(tbl, [key], jnp.maximum(cur, v))
            chk = plsc.load_gather(tbl, [key])
            lost = jnp.where(chk < v, 1, 0).astype(jnp.int32)

            key_s[pl.ds(off, 16)] = key
            v_s[pl.ds(off, 16)] = v
            m_s[pl.ds(off, 16)] = lost
            pend = pend | lost

            xt = xt + x * t
            cnt = cnt + t
            pr = pr + t * jnp.maximum(1.0 - x, 0.0)
            nr = nr + (1.0 - t) * jnp.maximum(x + 1.0, 0.0)
        return xt, cnt, pr, nr, pend

    xt, cnt, pr, nr, pend = lax.fori_loop(
        0, _NV // 4, body, (zero16, zero16, zero16, zero16, izero16))

    def rcond(p):
        return jnp.max(p) > 0

    def rbody(_p):
        def rinner(i, acc):
            for q in range(4):
                off = pl.multiple_of(i * 64, 64) + q * 16
                m = m_s[pl.ds(off, 16)] > 0
                key = key_s[pl.ds(off, 16)]
                v = v_s[pl.ds(off, 16)]
                tbl = tables[q]
                cur = plsc.load_gather(tbl, [key], mask=m)
                plsc.store_scatter(tbl, [key], jnp.maximum(cur, v), mask=m)
                chk = plsc.load_gather(tbl, [key], mask=m)
                lost = jnp.where(m & (chk < v), 1, 0).astype(jnp.int32)
                m_s[pl.ds(off, 16)] = lost
                acc = acc | lost
            return acc

        return lax.fori_loop(0, _NV // 4, rinner, izero16)

    lax.while_loop(rcond, rbody, pend)

    def merge_body(i, c):
        off = pl.multiple_of(i * 16, 16)
        a = jnp.maximum(tbl0[pl.ds(off, 16)], tbl1[pl.ds(off, 16)])
        b = jnp.maximum(tbl2[pl.ds(off, 16)], tbl3[pl.ds(off, 16)])
        tbl0[pl.ds(off, 16)] = jnp.maximum(a, b)
        return c

    lax.fori_loop(0, _TBL // 16, merge_body, 0)

    part_v[pl.ds(0, 16)] = xt
    part_v[pl.ds(16, 16)] = cnt
    part_v[pl.ds(32, 16)] = pr
    part_v[pl.ds(48, 16)] = nr

    pltpu.sync_copy(tbl0, tbl_out.at[wid])
    pltpu.sync_copy(part_v, part_out.at[wid])


_sc_call = functools.partial(
    pl.kernel,
    out_type=(
        jax.ShapeDtypeStruct((_NW, _TBL), jnp.float32),
        jax.ShapeDtypeStruct((_NW, 64), jnp.float32),
    ),
    mesh=plsc.VectorSubcoreMesh(core_axis_name="c", subcore_axis_name="s"),
    compiler_params=pltpu.CompilerParams(needs_layout_passes=False),
    scratch_types=[
        pltpu.VMEM((_CHUNK,), jnp.float32),
        pltpu.VMEM((_CHUNK,), jnp.int32),
        pltpu.VMEM((_CHUNK,), jnp.int32),
        pltpu.VMEM((_NPACK,), jnp.int32),
        pltpu.VMEM((_TBL,), jnp.float32),
        pltpu.VMEM((_TBL,), jnp.float32),
        pltpu.VMEM((_TBL,), jnp.float32),
        pltpu.VMEM((_TBL,), jnp.float32),
        pltpu.VMEM((_CHUNK,), jnp.int32),
        pltpu.VMEM((_CHUNK,), jnp.float32),
        pltpu.VMEM((_CHUNK,), jnp.int32),
        pltpu.VMEM((64,), jnp.float32),
        pltpu.SemaphoreType.DMA,
    ],
)(_sc_body)


def _tc_body(x_ref, tbl_ref, part_ref, out_ref):
    x = x_ref[...]
    bce_base = jnp.sum(jnp.maximum(x, 0.0) + jnp.log1p(jnp.exp(-jnp.abs(x))))

    tbl = jnp.max(tbl_ref[...], axis=0, keepdims=True)
    max_neg = tbl[:, :_B]
    neg_min_pos = tbl[:, _B:]
    ninf = jnp.float32(-jnp.inf)
    valid = (max_neg > ninf) & (neg_min_pos > ninf)
    samp = jnp.where(valid, jnp.maximum(_MARGIN + max_neg + neg_min_pos, 0.0), 0.0)
    num_valid = jnp.sum(valid.astype(jnp.float32))
    samp_sum = jnp.sum(samp)

    p = part_ref[...]
    s_xt = jnp.sum(p[:, 0:16])
    pos_cnt = jnp.sum(p[:, 16:32])
    pr = jnp.sum(p[:, 32:48])
    nr = jnp.sum(p[:, 48:64])

    n_valid = jnp.float32(_N)
    neg_cnt = n_valid - pos_cnt
    bce_loss = (bce_base - s_xt) / n_valid * _SCALE_LOSS
    avg_valid = n_valid / jnp.maximum(num_valid, 1.0)
    temperature = _TEMP_BASE / jnp.maximum(avg_valid, 1.0)
    contrastive = samp_sum * temperature
    sep = jnp.where(pos_cnt > 0.0, pr / jnp.maximum(pos_cnt, 1.0), 0.0)
    sep = sep + jnp.where(neg_cnt > 0.0, nr / jnp.maximum(neg_cnt, 1.0), 0.0)
    out_ref[0, 0] = bce_loss + contrastive + _SEP_W * sep


def kernel(logits, labels, batch_indices, label_ids):
    packed = pl.pallas_call(
        _pack_body,
        out_shape=jax.ShapeDtypeStruct((_B // 4, _L), jnp.int32),
    )(labels).reshape(-1)
    x_flat = logits.reshape(-1)

    tbl_all, part_all = _sc_call(x_flat, batch_indices, label_ids, packed)

    out = pl.pallas_call(
        _tc_body,
        out_shape=jax.ShapeDtypeStruct((1, 1), jnp.float32),
        out_specs=pl.BlockSpec(memory_space=pltpu.SMEM),
    )(x_flat, tbl_all, part_all)
    return out.reshape(())


# speedup vs baseline: 9.8835x; 1.1246x over previous
import functools

import jax
import jax.numpy as jnp
from jax import lax
from jax.experimental import pallas as pl
from jax.experimental.pallas import tpu as pltpu
from jax.experimental.pallas import tpu_sc as plsc

_B = 4096
_L = 50
_N = _B * _L
_NW = 32
_CHUNK = _N // _NW
_NV = _CHUNK // 16
_NPACK = _N // 32
_TBL = 2 * _B

_MARGIN = 0.5
_SCALE_LOSS = 10.0
_TEMP_BASE = 10.0
_SEP_W = 0.5


def _pre_body(lab_ref, bi_ref, li_ref, pk_ref, fused_ref):
    o = lab_ref[0:128, :]
    for w in range(1, 32):
        o = o | (lab_ref[128 * w:128 * (w + 1), :] << w)
    pk_ref[...] = o

    bi = bi_ref[...]
    li = li_ref[...]
    word = (bi & 127) * _L + li - 1
    sh = lax.shift_right_logical(bi, 7)
    fused_ref[...] = word | (sh << 13) | (bi << 18)


def _sc_body(logits_hbm, fused_hbm, packed_hbm,
             tbl_out, part_out,
             x_v, f_v, pk_v,
             tbl0, tbl1, tbl2, tbl3,
             key_s, v_s, part_v,
             sem):
    wid = lax.axis_index("s") * 2 + lax.axis_index("c")
    base = pl.multiple_of(wid * _CHUNK, _CHUNK)

    cx = pltpu.async_copy(logits_hbm.at[pl.ds(base, _CHUNK)], x_v, sem)
    cf = pltpu.async_copy(fused_hbm.at[pl.ds(base, _CHUNK)], f_v, sem)
    cp = pltpu.async_copy(packed_hbm, pk_v, sem)

    neg_inf16 = jnp.full((16,), -jnp.inf, dtype=jnp.float32)

    def init_body(i, c):
        off = pl.multiple_of(i * 16, 16)
        tbl0[pl.ds(off, 16)] = neg_inf16
        tbl1[pl.ds(off, 16)] = neg_inf16
        tbl2[pl.ds(off, 16)] = neg_inf16
        tbl3[pl.ds(off, 16)] = neg_inf16
        return c

    lax.fori_loop(0, _TBL // 16, init_body, 0)

    cx.wait()
    cf.wait()
    cp.wait()

    zero16 = jnp.zeros((16,), dtype=jnp.float32)
    tables = (tbl0, tbl1, tbl2, tbl3)

    def body(i, carry):
        accs = list(carry)
        for q in range(4):
            off = pl.multiple_of(i * 64, 64) + q * 16
            x = x_v[pl.ds(off, 16)]
            f = f_v[pl.ds(off, 16)]
            word = f & 0x1FFF
            sh = lax.shift_right_logical(f, 13) & 31
            bi = lax.shift_right_logical(f, 18)
            g = plsc.load_gather(pk_v, [word])
            tb = lax.shift_right_logical(g, sh) & 1
            t = tb.astype(jnp.float32)
            key = bi + lax.shift_left(tb, 12)
            v = x * (1.0 - 2.0 * t)

            tbl = tables[q]
            cur = plsc.load_gather(tbl, [key])
            plsc.store_scatter(tbl, [key], jnp.maximum(cur, v))
            key_s[pl.ds(off, 16)] = key
            v_s[pl.ds(off, 16)] = v

            r = jnp.maximum(1.0 - v, 0.0)
            tr = t * r
            xt, cnt, rs, trs = accs[4 * q:4 * q + 4]
            accs[4 * q:4 * q + 4] = (xt + x * t, cnt + t, rs + r, trs + tr)
        return tuple(accs)

    accs = lax.fori_loop(0, _NV // 4, body, (zero16,) * 16)
    xt = accs[0] + accs[4] + accs[8] + accs[12]
    cnt = accs[1] + accs[5] + accs[9] + accs[13]
    rs = accs[2] + accs[6] + accs[10] + accs[14]
    trs = accs[3] + accs[7] + accs[11] + accs[15]

    izero16 = jnp.zeros((16,), dtype=jnp.int32)

    def rcond(p):
        return jnp.max(p) > 0

    def rbody(_p):
        def rinner(i, acc):
            for q in range(4):
                off = pl.multiple_of(i * 64, 64) + q * 16
                key = key_s[pl.ds(off, 16)]
                v = v_s[pl.ds(off, 16)]
                tbl = tables[q]
                chk = plsc.load_gather(tbl, [key])
                m = chk < v
                plsc.store_scatter(tbl, [key], v, mask=m)
                chk2 = plsc.load_gather(tbl, [key], mask=m)
                acc = acc | jnp.where(m & (chk2 < v), 1, 0).astype(jnp.int32)
            return acc

        return lax.fori_loop(0, _NV // 4, rinner, izero16)

    lax.while_loop(rcond, rbody, jnp.ones((16,), dtype=jnp.int32))

    def merge_body(i, c):
        off = pl.multiple_of(i * 16, 16)
        a = jnp.maximum(tbl0[pl.ds(off, 16)], tbl1[pl.ds(off, 16)])
        b = jnp.maximum(tbl2[pl.ds(off, 16)], tbl3[pl.ds(off, 16)])
        tbl0[pl.ds(off, 16)] = jnp.maximum(a, b)
        return c

    lax.fori_loop(0, _TBL // 16, merge_body, 0)

    part_v[pl.ds(0, 16)] = xt
    part_v[pl.ds(16, 16)] = cnt
    part_v[pl.ds(32, 16)] = rs
    part_v[pl.ds(48, 16)] = trs

    pltpu.sync_copy(tbl0, tbl_out.at[wid])
    pltpu.sync_copy(part_v, part_out.at[wid])


_sc_call = functools.partial(
    pl.kernel,
    out_type=(
        jax.ShapeDtypeStruct((_NW, _TBL), jnp.float32),
        jax.ShapeDtypeStruct((_NW, 64), jnp.float32),
    ),
    mesh=plsc.VectorSubcoreMesh(core_axis_name="c", subcore_axis_name="s"),
    compiler_params=pltpu.CompilerParams(needs_layout_passes=False),
    scratch_types=[
        pltpu.VMEM((_CHUNK,), jnp.float32),
        pltpu.VMEM((_CHUNK,), jnp.int32),
        pltpu.VMEM((_NPACK,), jnp.int32),
        pltpu.VMEM((_TBL,), jnp.float32),
        pltpu.VMEM((_TBL,), jnp.float32),
        pltpu.VMEM((_TBL,), jnp.float32),
        pltpu.VMEM((_TBL,), jnp.float32),
        pltpu.VMEM((_CHUNK,), jnp.int32),
        pltpu.VMEM((_CHUNK,), jnp.float32),
        pltpu.VMEM((64,), jnp.float32),
        pltpu.SemaphoreType.DMA,
    ],
)(_sc_body)


def _bce_body(x_ref, out_ref):
    x = x_ref[...]
    out_ref[0, 0] = jnp.sum(jnp.maximum(x, 0.0) + jnp.log1p(jnp.exp(-jnp.abs(x))))


def _fin_body(tbl_ref, part_ref, bce_ref, out_ref):
    tbl = jnp.max(tbl_ref[...], axis=0, keepdims=True)
    max_neg = tbl[:, :_B]
    neg_min_pos = tbl[:, _B:]
    ninf = jnp.float32(-jnp.inf)
    valid = (max_neg > ninf) & (neg_min_pos > ninf)
    samp = jnp.where(valid, jnp.maximum(_MARGIN + max_neg + neg_min_pos, 0.0), 0.0)
    num_valid = jnp.sum(valid.astype(jnp.float32))
    samp_sum = jnp.sum(samp)

    p = part_ref[...]
    s_xt = jnp.sum(p[:, 0:16])
    pos_cnt = jnp.sum(p[:, 16:32])
    rs = jnp.sum(p[:, 32:48])
    trs = jnp.sum(p[:, 48:64])
    pr = trs
    nr = rs - trs

    n_valid = jnp.float32(_N)
    neg_cnt = n_valid - pos_cnt
    bce_loss = (bce_ref[0, 0] - s_xt) / n_valid * _SCALE_LOSS
    avg_valid = n_valid / jnp.maximum(num_valid, 1.0)
    temperature = _TEMP_BASE / jnp.maximum(avg_valid, 1.0)
    contrastive = samp_sum * temperature
    sep = jnp.where(pos_cnt > 0.0, pr / jnp.maximum(pos_cnt, 1.0), 0.0)
    sep = sep + jnp.where(neg_cnt > 0.0, nr / jnp.maximum(neg_cnt, 1.0), 0.0)
    out_ref[0, 0] = bce_loss + contrastive + _SEP_W * sep


def kernel(logits, labels, batch_indices, label_ids):
    pk, fused = pl.pallas_call(
        _pre_body,
        out_shape=(
            jax.ShapeDtypeStruct((128, _L), jnp.int32),
            jax.ShapeDtypeStruct((_N,), jnp.int32),
        ),
    )(labels, batch_indices, label_ids)
    x_flat = logits.reshape(-1)

    tbl_all, part_all = _sc_call(x_flat, fused, pk.reshape(-1))

    bce = pl.pallas_call(
        _bce_body,
        out_shape=jax.ShapeDtypeStruct((1, 1), jnp.float32),
        out_specs=pl.BlockSpec(memory_space=pltpu.SMEM),
    )(x_flat)

    out = pl.pallas_call(
        _fin_body,
        out_shape=jax.ShapeDtypeStruct((1, 1), jnp.float32),
        out_specs=pl.BlockSpec(memory_space=pltpu.SMEM),
        in_specs=[
            pl.BlockSpec(),
            pl.BlockSpec(),
            pl.BlockSpec(memory_space=pltpu.SMEM),
        ],
    )(tbl_all, part_all, bce)
    return out.reshape(())
